# baseline (speedup 1.0000x reference)
import functools

import jax
import jax.numpy as jnp
from jax import lax
from jax.experimental import pallas as pl
from jax.experimental.pallas import tpu as pltpu
from jax.experimental.pallas import tpu_sc as plsc

VOCAB = 1000000
EMBED = 64
B = 16384
L = 50

try:
    _info = plsc.get_sparse_core_info()
    _NC, _NS = _info.num_cores, _info.num_subcores
except Exception:
    _NC, _NS = 2, 16
_NW = _NC * _NS
_BPW = B // _NW


def _bag_body(text_hbm, emb_hbm, out_hbm, raw_v, idx_v, acc_v, sem):
    wid = lax.axis_index("s") * _NC + lax.axis_index("c")
    base = wid * _BPW
    half = _BPW // 2

    lanes = lax.iota(jnp.int32, 16)

    def stage_half(h, carry):
        pltpu.sync_copy(text_hbm.at[pl.ds(base + h * half, half), :], raw_v)

        def transp_j(j, carry1):
            col = jnp.full((16,), 0, jnp.int32) + j

            def transp_g(g, carry2):
                vals = plsc.load_gather(raw_v, [lanes + g * 16, col])
                idx_v[pl.ds(j * _BPW + h * half + g * 16, 16)] = vals
                return carry2

            lax.fori_loop(0, half // 16, transp_g, 0)
            return carry1

        lax.fori_loop(0, L, transp_j, 0)
        return carry

    lax.fori_loop(0, 2, stage_half, 0)

    pltpu.async_copy(emb_hbm.at[idx_v.at[pl.ds(0, _BPW)]], acc_v, sem).wait()

    def body(j, carry):
        pltpu.async_copy(
            emb_hbm.at[idx_v.at[pl.ds(j * _BPW, _BPW)]], acc_v, sem, add=True
        ).wait()
        return carry

    lax.fori_loop(1, L, body, 0)
    pltpu.sync_copy(acc_v.at[:, pl.ds(0, EMBED)], out_hbm.at[pl.ds(base, _BPW)])


@jax.jit
def _bag(text2d, emb):
    mesh = plsc.VectorSubcoreMesh(core_axis_name="c", subcore_axis_name="s")
    return pl.kernel(
        _bag_body,
        out_type=jax.ShapeDtypeStruct((B, EMBED), jnp.float32),
        mesh=mesh,
        scratch_types=[
            pltpu.VMEM((_BPW // 2, 128), jnp.int32),
            pltpu.VMEM((L * _BPW,), jnp.int32),
            pltpu.VMEM((_BPW, 128), jnp.float32),
            pltpu.SemaphoreType.DMA,
        ],
        compiler_params=pltpu.CompilerParams(
            use_tc_tiling_on_sc=False, needs_layout_passes=False
        ),
    )(text2d, emb)


_BLK = 4096


def _mlp_body(x_ref, w1_ref, b1_ref, w2_ref, b2_ref, w3_ref, b3_ref, o_ref):
    x = x_ref[...] * (1.0 / L)
    h = jnp.dot(x, w1_ref[...], preferred_element_type=jnp.float32) + b1_ref[...]
    h = jnp.maximum(h, 0.0)
    h = jnp.dot(h, w2_ref[...], preferred_element_type=jnp.float32) + b2_ref[...]
    h = jnp.maximum(h, 0.0)
    o_ref[...] = (
        jnp.dot(h, w3_ref[...], preferred_element_type=jnp.float32) + b3_ref[...]
    )


@jax.jit
def _mlp(sums, w1t, b1, w2t, b2, w3t, b3):
    grid = (B // _BLK,)
    return pl.pallas_call(
        _mlp_body,
        grid=grid,
        in_specs=[
            pl.BlockSpec((_BLK, EMBED), lambda i: (i, 0)),
            pl.BlockSpec((EMBED, EMBED), lambda i: (0, 0)),
            pl.BlockSpec((1, EMBED), lambda i: (0, 0)),
            pl.BlockSpec((EMBED, EMBED // 2), lambda i: (0, 0)),
            pl.BlockSpec((1, EMBED // 2), lambda i: (0, 0)),
            pl.BlockSpec((EMBED // 2, 1), lambda i: (0, 0)),
            pl.BlockSpec((1, 1), lambda i: (0, 0)),
        ],
        out_specs=pl.BlockSpec((_BLK, 1), lambda i: (i, 0)),
        out_shape=jax.ShapeDtypeStruct((B, 1), jnp.float32),
    )(sums, w1t, b1, w2t, b2, w3t, b3)


def kernel(text, emb, W1, b1, W2, b2, W3, b3):
    text_p = jnp.pad(text.astype(jnp.int32), ((0, 0), (0, 128 - L)))
    emb_p = jnp.pad(emb, ((0, 0), (0, 128 - EMBED)))
    sums = _bag(text_p, emb_p)
    out = _mlp(
        sums,
        jnp.transpose(W1),
        b1.reshape(1, EMBED),
        jnp.transpose(W2),
        b2.reshape(1, EMBED // 2),
        jnp.transpose(W3),
        b3.reshape(1, 1),
    )
    return jnp.squeeze(out, axis=-1)


# speedup vs baseline: 1.0782x; 1.0782x over previous
import functools

import jax
import jax.numpy as jnp
from jax import lax
from jax.experimental import pallas as pl
from jax.experimental.pallas import tpu as pltpu
from jax.experimental.pallas import tpu_sc as plsc

VOCAB = 1000000
EMBED = 64
B = 16384
L = 50

try:
    _info = plsc.get_sparse_core_info()
    _NC, _NS = _info.num_cores, _info.num_subcores
except Exception:
    _NC, _NS = 2, 16
_NW = _NC * _NS
_BPW = B // _NW


_DEPTH = 4


def _bag_body(text_hbm, emb_hbm, out_hbm, raw_v, idx_v, acc_v, sem):
    wid = lax.axis_index("s") * _NC + lax.axis_index("c")
    base = wid * _BPW

    pltpu.sync_copy(text_hbm.at[pl.ds(base, _BPW), :], raw_v)

    lanes = lax.iota(jnp.int32, 16)

    def transp_j(j):
        col = jnp.full((16,), 0, jnp.int32) + j

        def transp_g(g, carry2):
            vals = plsc.load_gather(raw_v, [lanes + g * 16, col])
            idx_v[pl.ds(j * _BPW + g * 16, 16)] = vals
            return carry2

        lax.fori_loop(0, _BPW // 16, transp_g, 0)

    def drain_one():
        pltpu.make_async_copy(emb_hbm.at[pl.ds(0, _BPW)], acc_v, sem).wait()

    transp_j(0)
    pltpu.async_copy(emb_hbm.at[idx_v.at[pl.ds(0, _BPW)]], acc_v, sem).wait()

    def body(j, carry):
        transp_j(j)
        pltpu.async_copy(
            emb_hbm.at[idx_v.at[pl.ds(j * _BPW, _BPW)]], acc_v, sem, add=True
        )

        @pl.when(j >= _DEPTH + 1)
        def _():
            drain_one()

        return carry

    lax.fori_loop(1, L, body, 0)
    for _ in range(_DEPTH):
        drain_one()
    pltpu.sync_copy(acc_v, out_hbm.at[pl.ds(base, _BPW)])


@jax.jit
def _bag(text2d, emb):
    mesh = plsc.VectorSubcoreMesh(core_axis_name="c", subcore_axis_name="s")
    return pl.kernel(
        _bag_body,
        out_type=jax.ShapeDtypeStruct((B, EMBED), jnp.float32),
        mesh=mesh,
        scratch_types=[
            pltpu.VMEM((_BPW, 128), jnp.int32),
            pltpu.VMEM((L * _BPW,), jnp.int32),
            pltpu.VMEM((_BPW, EMBED), jnp.float32),
            pltpu.SemaphoreType.DMA,
        ],
        compiler_params=pltpu.CompilerParams(
            use_tc_tiling_on_sc=False, needs_layout_passes=False
        ),
    )(text2d, emb)


_BLK = 4096


def _mlp_body(x_ref, w1_ref, b1_ref, w2_ref, b2_ref, w3_ref, b3_ref, o_ref):
    x = x_ref[...] * (1.0 / L)
    h = jnp.dot(x, w1_ref[...], preferred_element_type=jnp.float32) + b1_ref[...]
    h = jnp.maximum(h, 0.0)
    h = jnp.dot(h, w2_ref[...], preferred_element_type=jnp.float32) + b2_ref[...]
    h = jnp.maximum(h, 0.0)
    o_ref[...] = (
        jnp.dot(h, w3_ref[...], preferred_element_type=jnp.float32) + b3_ref[...]
    )


@jax.jit
def _mlp(sums, w1t, b1, w2t, b2, w3t, b3):
    grid = (B // _BLK,)
    return pl.pallas_call(
        _mlp_body,
        grid=grid,
        in_specs=[
            pl.BlockSpec((_BLK, EMBED), lambda i: (i, 0)),
            pl.BlockSpec((EMBED, EMBED), lambda i: (0, 0)),
            pl.BlockSpec((1, EMBED), lambda i: (0, 0)),
            pl.BlockSpec((EMBED, EMBED // 2), lambda i: (0, 0)),
            pl.BlockSpec((1, EMBED // 2), lambda i: (0, 0)),
            pl.BlockSpec((EMBED // 2, 1), lambda i: (0, 0)),
            pl.BlockSpec((1, 1), lambda i: (0, 0)),
        ],
        out_specs=pl.BlockSpec((_BLK, 1), lambda i: (i, 0)),
        out_shape=jax.ShapeDtypeStruct((B, 1), jnp.float32),
    )(sums, w1t, b1, w2t, b2, w3t, b3)


def kernel(text, emb, W1, b1, W2, b2, W3, b3):
    text_p = jnp.pad(text.astype(jnp.int32), ((0, 0), (0, 128 - L)))
    sums = _bag(text_p, emb)
    out = _mlp(
        sums,
        jnp.transpose(W1),
        b1.reshape(1, EMBED),
        jnp.transpose(W2),
        b2.reshape(1, EMBED // 2),
        jnp.transpose(W3),
        b3.reshape(1, 1),
    )
    return jnp.squeeze(out, axis=-1)


# speedup vs baseline: 1.0853x; 1.0066x over previous
import functools

import jax
import jax.numpy as jnp
from jax import lax
from jax.experimental import pallas as pl
from jax.experimental.pallas import tpu as pltpu
from jax.experimental.pallas import tpu_sc as plsc

VOCAB = 1000000
EMBED = 64
B = 16384
L = 50

try:
    _info = plsc.get_sparse_core_info()
    _NC, _NS = _info.num_cores, _info.num_subcores
except Exception:
    _NC, _NS = 2, 16
_NW = _NC * _NS
_BPW = B // _NW


_DEPTH = 8


def _bag_body(text_hbm, emb_hbm, out_hbm, raw_v, idx_v, acc_v, sem):
    wid = lax.axis_index("s") * _NC + lax.axis_index("c")
    base = wid * _BPW

    pltpu.sync_copy(text_hbm.at[pl.ds(base, _BPW), :], raw_v)

    lanes = lax.iota(jnp.int32, 16)

    def transp_j(j):
        col = jnp.full((16,), 0, jnp.int32) + j

        def transp_g(g, carry2):
            vals = plsc.load_gather(raw_v, [lanes + g * 16, col])
            idx_v[pl.ds(j * _BPW + g * 16, 16)] = vals
            return carry2

        lax.fori_loop(0, _BPW // 16, transp_g, 0)

    def drain_one():
        pltpu.make_async_copy(emb_hbm.at[pl.ds(0, _BPW)], acc_v, sem).wait()

    transp_j(0)
    pltpu.async_copy(emb_hbm.at[idx_v.at[pl.ds(0, _BPW)]], acc_v, sem).wait()

    def body(j, carry):
        transp_j(j)
        pltpu.async_copy(
            emb_hbm.at[idx_v.at[pl.ds(j * _BPW, _BPW)]], acc_v, sem, add=True
        )

        @pl.when(j >= _DEPTH + 1)
        def _():
            drain_one()

        return carry

    lax.fori_loop(1, L, body, 0)
    for _ in range(_DEPTH):
        drain_one()
    pltpu.sync_copy(acc_v, out_hbm.at[pl.ds(base, _BPW)])


@jax.jit
def _bag(text2d, emb):
    mesh = plsc.VectorSubcoreMesh(core_axis_name="c", subcore_axis_name="s")
    return pl.kernel(
        _bag_body,
        out_type=jax.ShapeDtypeStruct((B, EMBED), jnp.float32),
        mesh=mesh,
        scratch_types=[
            pltpu.VMEM((_BPW, 128), jnp.int32),
            pltpu.VMEM((L * _BPW,), jnp.int32),
            pltpu.VMEM((_BPW, EMBED), jnp.float32),
            pltpu.SemaphoreType.DMA,
        ],
        compiler_params=pltpu.CompilerParams(
            use_tc_tiling_on_sc=False, needs_layout_passes=False
        ),
    )(text2d, emb)


_BLK = 4096


def _mlp_body(x_ref, w1_ref, b1_ref, w2_ref, b2_ref, w3_ref, b3_ref, o_ref):
    x = x_ref[...] * (1.0 / L)
    h = jnp.dot(x, w1_ref[...], preferred_element_type=jnp.float32) + b1_ref[...]
    h = jnp.maximum(h, 0.0)
    h = jnp.dot(h, w2_ref[...], preferred_element_type=jnp.float32) + b2_ref[...]
    h = jnp.maximum(h, 0.0)
    o_ref[...] = (
        jnp.dot(h, w3_ref[...], preferred_element_type=jnp.float32) + b3_ref[...]
    )


@jax.jit
def _mlp(sums, w1t, b1, w2t, b2, w3t, b3):
    grid = (B // _BLK,)
    return pl.pallas_call(
        _mlp_body,
        grid=grid,
        in_specs=[
            pl.BlockSpec((_BLK, EMBED), lambda i: (i, 0)),
            pl.BlockSpec((EMBED, EMBED), lambda i: (0, 0)),
            pl.BlockSpec((1, EMBED), lambda i: (0, 0)),
            pl.BlockSpec((EMBED, EMBED // 2), lambda i: (0, 0)),
            pl.BlockSpec((1, EMBED // 2), lambda i: (0, 0)),
            pl.BlockSpec((EMBED // 2, 1), lambda i: (0, 0)),
            pl.BlockSpec((1, 1), lambda i: (0, 0)),
        ],
        out_specs=pl.BlockSpec((_BLK, 1), lambda i: (i, 0)),
        out_shape=jax.ShapeDtypeStruct((B, 1), jnp.float32),
    )(sums, w1t, b1, w2t, b2, w3t, b3)


def kernel(text, emb, W1, b1, W2, b2, W3, b3):
    text_p = jnp.pad(text.astype(jnp.int32), ((0, 0), (0, 128 - L)))
    sums = _bag(text_p, emb)
    out = _mlp(
        sums,
        jnp.transpose(W1),
        b1.reshape(1, EMBED),
        jnp.transpose(W2),
        b2.reshape(1, EMBED // 2),
        jnp.transpose(W3),
        b3.reshape(1, 1),
    )
    return jnp.squeeze(out, axis=-1)
